# baseline (speedup 1.0000x reference)
import functools

import jax
import jax.numpy as jnp
from jax import lax
from jax.experimental import pallas as pl
from jax.experimental.pallas import tpu as pltpu
from jax.experimental.pallas import tpu_sc as plsc

N = 10000
N_PAD = 10240
E = 160000
E_PAD = 163840
ER = E_PAD // 128
F_IN = 256
HID = 512
DV = 256

NC = 2
NS = 16
LANES = 16
K = 128

ROWS_PER_TILE = N_PAD // NS

DROWS = ER // (NC * NS)
KA = 64
CH = E_PAD // (NS * 2 * KA)
NB = 4
LOOK = 3


def _mesh():
    return plsc.VectorSubcoreMesh(core_axis_name="c", subcore_axis_name="s")


@functools.cache
def _get_deg_counts():
    return functools.partial(
        pl.kernel,
        out_type=jax.ShapeDtypeStruct((NC, N_PAD, 128), jnp.float32),
        mesh=_mesh(),
        scratch_types=[
            pltpu.VMEM((DROWS, K), jnp.int32),
            pltpu.VMEM((K, 128), jnp.float32),
            pltpu.VMEM_SHARED((N_PAD, 128), jnp.float32),
            pltpu.SemaphoreType.DMA,
        ],
    )(_deg_body)


def _deg_body(dst_hbm, ones_hbm, zeros_hbm, out_hbm, dst_v, ones_v, acc, sem):
    c = lax.axis_index("c")
    s = lax.axis_index("s")
    r0 = s * ROWS_PER_TILE
    pltpu.sync_copy(zeros_hbm.at[pl.ds(r0, ROWS_PER_TILE)],
                    acc.at[pl.ds(r0, ROWS_PER_TILE)])
    pltpu.sync_copy(ones_hbm, ones_v)
    pltpu.sync_copy(dst_hbm.at[pl.ds((c * NS + s) * DROWS, DROWS)], dst_v)
    plsc.subcore_barrier()

    def fire(i, carry):
        pltpu.async_copy(ones_v, acc.at[dst_v.at[i]], sem, add=True)
        return carry

    lax.fori_loop(0, DROWS, fire, 0)

    def drain(i, carry):
        pltpu.make_async_copy(ones_v, acc.at[dst_v.at[i]], sem).wait()
        return carry

    lax.fori_loop(0, DROWS, drain, 0)
    plsc.subcore_barrier()
    pltpu.sync_copy(acc.at[pl.ds(r0, ROWS_PER_TILE)],
                    out_hbm.at[c, pl.ds(r0, ROWS_PER_TILE)])


@functools.cache
def _make_scatter(Q):
    P = Q // NC

    @functools.partial(
        pl.kernel,
        out_type=jax.ShapeDtypeStruct((Q, N_PAD, 128), jnp.float32),
        mesh=_mesh(),
        scratch_types=(
            [
                pltpu.VMEM((CH // 2, 128), jnp.int32),
                pltpu.VMEM((CH // 2, 128), jnp.int32),
            ]
            + [pltpu.VMEM((KA,), jnp.int32) for _ in range(NB)]
            + [pltpu.VMEM((KA,), jnp.int32) for _ in range(NB)]
            + [pltpu.VMEM((KA, 128), jnp.float32) for _ in range(NB)]
            + [pltpu.VMEM_SHARED((N_PAD, 128), jnp.float32)]
            + [pltpu.SemaphoreType.DMA for _ in range(2 * NB)]
        ),
    )
    def _scatter(u_hbm, src_hbm, dst_hbm, out_hbm,
                 src_v, dst_v, ib0, ib1, ib2, ib3, db0, db1, db2, db3,
                 r0_v, r1_v, r2_v, r3_v, acc,
                 g0, g1, g2, g3, s0, s1, s2, s3):
        ibuf = [ib0, ib1, ib2, ib3]
        dbuf = [db0, db1, db2, db3]
        rows = [r0_v, r1_v, r2_v, r3_v]
        gsem = [g0, g1, g2, g3]
        ssem = [s0, s1, s2, s3]
        c = lax.axis_index("c")
        s = lax.axis_index("s")
        r0 = s * ROWS_PER_TILE

        def prep(buf, ev, row, col, add):
            for t in range(KA // LANES):
                buf[pl.ds(t * LANES, LANES)] = (
                    ev[row, pl.ds(col + t * LANES, LANES)] + add)

        for p in range(P):
            q = p * NC + c
            qn = q * N_PAD
            pltpu.sync_copy(u_hbm.at[pl.ds(qn + r0, ROWS_PER_TILE)],
                            acc.at[pl.ds(r0, ROWS_PER_TILE)])
            plsc.subcore_barrier()
            for h in range(2):
                e0 = s * (CH // 2) * 2 + h * (CH // 2)
                pltpu.sync_copy(src_hbm.at[pl.ds(e0, CH // 2)], src_v)
                pltpu.sync_copy(dst_hbm.at[pl.ds(e0, CH // 2)], dst_v)

                for j in range(LOOK):
                    prep(ibuf[j], src_v, j // 2, KA * (j % 2), qn)
                    pltpu.async_copy(u_hbm.at[ibuf[j]], rows[j], gsem[j])

                def quad(g, carry):
                    for j in range(NB):
                        ci = NB * g + j
                        row = 2 * g + j // 2
                        col = KA * (j % 2)
                        pltpu.make_async_copy(u_hbm.at[ibuf[j]],
                                              rows[j], gsem[j]).wait()
                        prep(dbuf[j], dst_v, row, col, 0)
                        pltpu.async_copy(rows[j], acc.at[dbuf[j]],
                                         ssem[j], add=True)
                        jn = (j + LOOK) % NB

                        rown = 2 * g + (j + LOOK) // 2
                        coln = KA * ((j + LOOK) % 2)

                        @pl.when(ci + LOOK < CH)
                        def _(ci=ci, jn=jn, rown=rown, coln=coln):
                            @pl.when(ci + LOOK >= NB)
                            def _():
                                pltpu.make_async_copy(rows[jn],
                                                      acc.at[dbuf[jn]],
                                                      ssem[jn]).wait()
                            prep(ibuf[jn], src_v, rown, coln, qn)
                            pltpu.async_copy(u_hbm.at[ibuf[jn]],
                                             rows[jn], gsem[jn])
                    return carry

                lax.fori_loop(0, CH // NB, quad, 0)
                for j in range(NB):
                    pltpu.make_async_copy(rows[j], acc.at[dbuf[j]],
                                          ssem[j]).wait()
            plsc.subcore_barrier()
            pltpu.sync_copy(acc.at[pl.ds(r0, ROWS_PER_TILE)],
                            out_hbm.at[q, pl.ds(r0, ROWS_PER_TILE)])

    return _scatter


BLK = 1024
GRID = N_PAD // BLK


def _prep_body(degp_ref, x_ref, u_ref, dinv_ref):
    deg = 1.0 + degp_ref[0, :, 0:1] + degp_ref[1, :, 0:1]
    dinv = lax.rsqrt(deg)
    dinv_ref[...] = dinv
    u = dinv * x_ref[...]
    u_ref[0] = u[:, :128]
    u_ref[1] = u[:, 128:]


def _mid_body(s_ref, dinv_ref, wp0_ref, bp0_ref, ws0_ref, bs0_ref,
              wp1_ref, ws1_ref, v_ref):
    dinv = dinv_ref[...]
    a0 = dinv * jnp.concatenate([s_ref[0], s_ref[1]], axis=1)
    hp = jnp.maximum(jnp.dot(a0, wp0_ref[...]) + bp0_ref[...], 0.0)
    hs = jnp.maximum(jnp.dot(a0, ws0_ref[...]) + bs0_ref[...], 0.0)
    vp = dinv * jnp.dot(hp, wp1_ref[...])
    vs = dinv * jnp.dot(hs, ws1_ref[...])
    v_ref[0] = vp[:, :128]
    v_ref[1] = vp[:, 128:]
    v_ref[2] = vs[:, :128]
    v_ref[3] = vs[:, 128:]


def _fin_body(s_ref, dinv_ref, bp1_ref, bs1_ref, wa_ref, va_ref,
              wc_ref, bc_ref, out_ref):
    dinv = dinv_ref[...]
    mup = dinv * jnp.concatenate([s_ref[0], s_ref[1]], axis=1) + bp1_ref[...]
    mus = dinv * jnp.concatenate([s_ref[2], s_ref[3]], axis=1) + bs1_ref[...]
    sp = jnp.dot(jnp.tanh(jnp.dot(mup, wa_ref[...])), va_ref[...])
    ss = jnp.dot(jnp.tanh(jnp.dot(mus, wa_ref[...])), va_ref[...])
    m = jnp.maximum(sp, ss)
    e0 = jnp.exp(sp - m)
    e1 = jnp.exp(ss - m)
    z = (e0 * mup + e1 * mus) / (e0 + e1)
    out_ref[...] = jnp.dot(z, wc_ref[...]) + bc_ref[...]


def _full(shape):
    nd = len(shape)
    return pl.BlockSpec(shape, lambda i, _nd=nd: (0,) * _nd)


def kernel(x, edge_index, Wp0, bp0, Wp1, bp1, Wp2, bp2,
           Ws0, bs0, Ws1, bs1, Ws2, bs2, Wa, va, Wc, bc):
    src = edge_index[0]
    dst = edge_index[1]
    src2 = jnp.concatenate(
        [src, jnp.zeros((E_PAD - E,), jnp.int32)]).reshape(ER, K)
    dst2 = jnp.concatenate(
        [dst, jnp.full((E_PAD - E,), N, jnp.int32)]).reshape(ER, K)

    degp = _get_deg_counts()(dst2,
                             jnp.ones((K, 128), jnp.float32),
                             jnp.zeros((N_PAD, 128), jnp.float32))
    x_pad = jnp.pad(x, ((0, N_PAD - N), (0, 0)))

    u3, dinv = pl.pallas_call(
        _prep_body,
        grid=(GRID,),
        in_specs=[
            pl.BlockSpec((NC, BLK, 128), lambda i: (0, i, 0)),
            pl.BlockSpec((BLK, F_IN), lambda i: (i, 0)),
        ],
        out_specs=[
            pl.BlockSpec((2, BLK, 128), lambda i: (0, i, 0)),
            pl.BlockSpec((BLK, 1), lambda i: (i, 0)),
        ],
        out_shape=[
            jax.ShapeDtypeStruct((2, N_PAD, 128), jnp.float32),
            jax.ShapeDtypeStruct((N_PAD, 1), jnp.float32),
        ],
    )(degp, x_pad)

    s2 = _make_scatter(2)(u3.reshape(2 * N_PAD, 128), src2, dst2)

    vcat = pl.pallas_call(
        _mid_body,
        grid=(GRID,),
        in_specs=[
            pl.BlockSpec((2, BLK, 128), lambda i: (0, i, 0)),
            pl.BlockSpec((BLK, 1), lambda i: (i, 0)),
            _full((F_IN, HID)),
            _full((1, HID)),
            _full((F_IN, HID)),
            _full((1, HID)),
            _full((HID, DV)),
            _full((HID, DV)),
        ],
        out_specs=pl.BlockSpec((4, BLK, 128), lambda i: (0, i, 0)),
        out_shape=jax.ShapeDtypeStruct((4, N_PAD, 128), jnp.float32),
    )(s2, dinv, Wp0, bp0.reshape(1, HID), Ws0, bs0.reshape(1, HID), Wp1, Ws1)

    s4 = _make_scatter(4)(vcat.reshape(4 * N_PAD, 128), src2, dst2)

    out = pl.pallas_call(
        _fin_body,
        grid=(GRID,),
        in_specs=[
            pl.BlockSpec((4, BLK, 128), lambda i: (0, i, 0)),
            pl.BlockSpec((BLK, 1), lambda i: (i, 0)),
            _full((1, DV)),
            _full((1, DV)),
            _full((DV, 128)),
            _full((128, 1)),
            _full((DV, 16)),
            _full((1, 16)),
        ],
        out_specs=pl.BlockSpec((BLK, 16), lambda i: (i, 0)),
        out_shape=jax.ShapeDtypeStruct((N_PAD, 16), jnp.float32),
    )(s4, dinv, bp1.reshape(1, DV), bs1.reshape(1, DV),
      Wa, va.reshape(128, 1), Wc, bc.reshape(1, 16))

    return out[:N]


# speedup vs baseline: 1.0948x; 1.0948x over previous
import functools

import jax
import jax.numpy as jnp
from jax import lax
from jax.experimental import pallas as pl
from jax.experimental.pallas import tpu as pltpu
from jax.experimental.pallas import tpu_sc as plsc

N = 10000
N_PAD = 10240
E = 160000
E_PAD = 163840
ER = E_PAD // 128
F_IN = 256
HID = 512
DV = 256

NC = 2
NS = 16
LANES = 16
K = 128

ROWS_PER_TILE = N_PAD // NS

DROWS = ER // (NC * NS)
KA = 64
CH = E_PAD // (NS * 2 * KA)
NB = 4
LOOK = 3


def _mesh():
    return plsc.VectorSubcoreMesh(core_axis_name="c", subcore_axis_name="s")


@functools.cache
def _get_deg_counts():
    return functools.partial(
        pl.kernel,
        out_type=jax.ShapeDtypeStruct((NC, N_PAD, 128), jnp.float32),
        mesh=_mesh(),
        scratch_types=[
            pltpu.VMEM((DROWS, K), jnp.int32),
            pltpu.VMEM((K, 128), jnp.float32),
            pltpu.VMEM_SHARED((N_PAD, 128), jnp.float32),
            pltpu.SemaphoreType.DMA,
        ],
    )(_deg_body)


def _deg_body(dst_hbm, ones_hbm, zeros_hbm, out_hbm, dst_v, ones_v, acc, sem):
    c = lax.axis_index("c")
    s = lax.axis_index("s")
    r0 = s * ROWS_PER_TILE
    pltpu.sync_copy(zeros_hbm.at[pl.ds(r0, ROWS_PER_TILE)],
                    acc.at[pl.ds(r0, ROWS_PER_TILE)])
    pltpu.sync_copy(ones_hbm, ones_v)
    pltpu.sync_copy(dst_hbm.at[pl.ds((c * NS + s) * DROWS, DROWS)], dst_v)
    plsc.subcore_barrier()

    def fire(i, carry):
        pltpu.async_copy(ones_v, acc.at[dst_v.at[i]], sem, add=True)
        return carry

    lax.fori_loop(0, DROWS, fire, 0)

    def drain(i, carry):
        pltpu.make_async_copy(ones_v, acc.at[dst_v.at[i]], sem).wait()
        return carry

    lax.fori_loop(0, DROWS, drain, 0)
    plsc.subcore_barrier()
    pltpu.sync_copy(acc.at[pl.ds(r0, ROWS_PER_TILE)],
                    out_hbm.at[c, pl.ds(r0, ROWS_PER_TILE)])


def _edge_block(u_hbm, src_hbm, dst_hbm, acc,
                src_v, dst_v, ibuf, dbuf, rows, gsem, ssem, e0, qn):
    pltpu.sync_copy(src_hbm.at[pl.ds(e0, CH // 2)], src_v)
    pltpu.sync_copy(dst_hbm.at[pl.ds(e0, CH // 2)], dst_v)

    def prep(buf, ev, row, col, add):
        for t in range(KA // LANES):
            buf[pl.ds(t * LANES, LANES)] = (
                ev[row, pl.ds(col + t * LANES, LANES)] + add)

    for j in range(LOOK):
        prep(ibuf[j], src_v, j // 2, KA * (j % 2), qn)
        pltpu.async_copy(u_hbm.at[ibuf[j]], rows[j], gsem[j])

    def quad(g, carry):
        for j in range(NB):
            ci = NB * g + j
            row = 2 * g + j // 2
            col = KA * (j % 2)
            pltpu.make_async_copy(u_hbm.at[ibuf[j]], rows[j], gsem[j]).wait()
            prep(dbuf[j], dst_v, row, col, 0)
            pltpu.async_copy(rows[j], acc.at[dbuf[j]], ssem[j], add=True)
            jn = (j + LOOK) % NB
            rown = 2 * g + (j + LOOK) // 2
            coln = KA * ((j + LOOK) % 2)

            @pl.when(ci + LOOK < CH)
            def _(ci=ci, jn=jn, rown=rown, coln=coln):
                @pl.when(ci + LOOK >= NB)
                def _():
                    pltpu.make_async_copy(rows[jn], acc.at[dbuf[jn]],
                                          ssem[jn]).wait()
                prep(ibuf[jn], src_v, rown, coln, qn)
                pltpu.async_copy(u_hbm.at[ibuf[jn]], rows[jn], gsem[jn])
        return carry

    lax.fori_loop(0, CH // NB, quad, 0)
    for j in range(NB):
        pltpu.make_async_copy(rows[j], acc.at[dbuf[j]], ssem[j]).wait()


def _sc_scratch():
    return (
        [
            pltpu.VMEM((CH // 2, 128), jnp.int32),
            pltpu.VMEM((CH // 2, 128), jnp.int32),
        ]
        + [pltpu.VMEM((KA,), jnp.int32) for _ in range(NB)]
        + [pltpu.VMEM((KA,), jnp.int32) for _ in range(NB)]
        + [pltpu.VMEM((KA, 128), jnp.float32) for _ in range(NB)]
        + [pltpu.VMEM_SHARED((N_PAD, 128), jnp.float32)]
        + [pltpu.SemaphoreType.DMA for _ in range(2 * NB)]
    )


@functools.cache
def _make_scatter(Q):
    P = Q // NC

    @functools.partial(
        pl.kernel,
        out_type=jax.ShapeDtypeStruct((Q, N_PAD, 128), jnp.float32),
        mesh=_mesh(),
        scratch_types=_sc_scratch(),
    )
    def _scatter(u_hbm, src_hbm, dst_hbm, out_hbm,
                 src_v, dst_v, ib0, ib1, ib2, ib3, db0, db1, db2, db3,
                 r0_v, r1_v, r2_v, r3_v, acc,
                 g0, g1, g2, g3, s0, s1, s2, s3):
        ibuf = [ib0, ib1, ib2, ib3]
        dbuf = [db0, db1, db2, db3]
        rows = [r0_v, r1_v, r2_v, r3_v]
        gsem = [g0, g1, g2, g3]
        ssem = [s0, s1, s2, s3]
        c = lax.axis_index("c")
        s = lax.axis_index("s")
        r0 = s * ROWS_PER_TILE
        for p in range(P):
            q = p * NC + c
            qn = q * N_PAD
            pltpu.sync_copy(u_hbm.at[pl.ds(qn + r0, ROWS_PER_TILE)],
                            acc.at[pl.ds(r0, ROWS_PER_TILE)])
            plsc.subcore_barrier()
            for h in range(2):
                _edge_block(u_hbm, src_hbm, dst_hbm, acc, src_v, dst_v,
                            ibuf, dbuf, rows, gsem, ssem,
                            s * CH + h * (CH // 2), qn)
            plsc.subcore_barrier()
            pltpu.sync_copy(acc.at[pl.ds(r0, ROWS_PER_TILE)],
                            out_hbm.at[q, pl.ds(r0, ROWS_PER_TILE)])

    return _scatter


@functools.cache
def _get_scatter_l2():
    @functools.partial(
        pl.kernel,
        out_type=jax.ShapeDtypeStruct((4, N_PAD, 128), jnp.float32),
        mesh=_mesh(),
        scratch_types=_sc_scratch(),
    )
    def _scatter3(w_hbm, src_hbm, dst_hbm, zeros_hbm, out_hbm,
                  src_v, dst_v, ib0, ib1, ib2, ib3, db0, db1, db2, db3,
                  r0_v, r1_v, r2_v, r3_v, acc,
                  g0, g1, g2, g3, s0, s1, s2, s3):
        ibuf = [ib0, ib1, ib2, ib3]
        dbuf = [db0, db1, db2, db3]
        rows = [r0_v, r1_v, r2_v, r3_v]
        gsem = [g0, g1, g2, g3]
        ssem = [s0, s1, s2, s3]
        c = lax.axis_index("c")
        s = lax.axis_index("s")
        r0 = s * ROWS_PER_TILE
        qn = c * N_PAD
        pltpu.sync_copy(w_hbm.at[pl.ds(qn + r0, ROWS_PER_TILE)],
                        acc.at[pl.ds(r0, ROWS_PER_TILE)])
        plsc.subcore_barrier()
        for h in range(2):
            _edge_block(w_hbm, src_hbm, dst_hbm, acc, src_v, dst_v,
                        ibuf, dbuf, rows, gsem, ssem,
                        s * CH + h * (CH // 2), qn)
        plsc.subcore_barrier()
        pltpu.sync_copy(acc.at[pl.ds(r0, ROWS_PER_TILE)],
                        out_hbm.at[c, pl.ds(r0, ROWS_PER_TILE)])
        qn2 = 2 * N_PAD

        @pl.when(c == 0)
        def _():
            pltpu.sync_copy(w_hbm.at[pl.ds(qn2 + r0, ROWS_PER_TILE)],
                            acc.at[pl.ds(r0, ROWS_PER_TILE)])

        @pl.when(c == 1)
        def _():
            pltpu.sync_copy(zeros_hbm.at[pl.ds(r0, ROWS_PER_TILE)],
                            acc.at[pl.ds(r0, ROWS_PER_TILE)])

        plsc.subcore_barrier()
        _edge_block(w_hbm, src_hbm, dst_hbm, acc, src_v, dst_v,
                    ibuf, dbuf, rows, gsem, ssem,
                    (c * NS + s) * (CH // 2), qn2)
        plsc.subcore_barrier()
        pltpu.sync_copy(acc.at[pl.ds(r0, ROWS_PER_TILE)],
                        out_hbm.at[2 + c, pl.ds(r0, ROWS_PER_TILE)])

    return _scatter3


BLK = 1024
GRID = N_PAD // BLK


def _prep_body(degp_ref, x_ref, u_ref, dinv_ref):
    deg = 1.0 + degp_ref[0, :, 0:1] + degp_ref[1, :, 0:1]
    dinv = lax.rsqrt(deg)
    dinv_ref[...] = dinv
    u = dinv * x_ref[...]
    u_ref[0] = u[:, :128]
    u_ref[1] = u[:, 128:]


def _wts_body(wp1_ref, ws1_ref, wa_ref, wc_ref, bp1_ref, bs1_ref,
              wpa_ref, wsa_ref, wpc_ref, wsc_ref,
              bpa_ref, bsa_ref, bpc_ref, bsc_ref):
    wpa_ref[...] = jnp.dot(wp1_ref[...], wa_ref[...])
    wsa_ref[...] = jnp.dot(ws1_ref[...], wa_ref[...])
    wpc_ref[...] = jnp.dot(wp1_ref[...], wc_ref[...])
    wsc_ref[...] = jnp.dot(ws1_ref[...], wc_ref[...])
    bpa_ref[...] = jnp.dot(bp1_ref[...], wa_ref[...])
    bsa_ref[...] = jnp.dot(bs1_ref[...], wa_ref[...])
    bpc_ref[...] = jnp.dot(bp1_ref[...], wc_ref[...])
    bsc_ref[...] = jnp.dot(bs1_ref[...], wc_ref[...])


def _mid_body(s_ref, dinv_ref, wp0_ref, bp0_ref, ws0_ref, bs0_ref,
              wpa_ref, wsa_ref, wpc_ref, wsc_ref, w_ref):
    dinv = dinv_ref[...]
    a0 = dinv * jnp.concatenate([s_ref[0], s_ref[1]], axis=1)
    hp = jnp.maximum(jnp.dot(a0, wp0_ref[...]) + bp0_ref[...], 0.0)
    hs = jnp.maximum(jnp.dot(a0, ws0_ref[...]) + bs0_ref[...], 0.0)
    w_ref[0] = dinv * jnp.dot(hp, wpa_ref[...])
    w_ref[1] = dinv * jnp.dot(hs, wsa_ref[...])
    w_ref[2] = jnp.concatenate(
        [dinv * jnp.dot(hp, wpc_ref[...]),
         dinv * jnp.dot(hs, wsc_ref[...]),
         jnp.zeros((hp.shape[0], 96), jnp.float32)], axis=1)


def _fin_body(s_ref, dinv_ref, bpa_ref, bsa_ref, bpc_ref, bsc_ref,
              va_ref, bc_ref, out_ref):
    dinv = dinv_ref[...]
    muap = dinv * s_ref[0] + bpa_ref[...]
    muas = dinv * s_ref[1] + bsa_ref[...]
    sp = jnp.dot(jnp.tanh(muap), va_ref[...])
    ss = jnp.dot(jnp.tanh(muas), va_ref[...])
    mc = dinv * (s_ref[2] + s_ref[3])
    mcp = mc[:, 0:16] + bpc_ref[...]
    mcs = mc[:, 16:32] + bsc_ref[...]
    m = jnp.maximum(sp, ss)
    e0 = jnp.exp(sp - m)
    e1 = jnp.exp(ss - m)
    out_ref[...] = (e0 * mcp + e1 * mcs) / (e0 + e1) + bc_ref[...]


def _full(shape):
    nd = len(shape)
    return pl.BlockSpec(shape, lambda i, _nd=nd: (0,) * _nd)


def kernel(x, edge_index, Wp0, bp0, Wp1, bp1, Wp2, bp2,
           Ws0, bs0, Ws1, bs1, Ws2, bs2, Wa, va, Wc, bc):
    src = edge_index[0]
    dst = edge_index[1]
    src2 = jnp.concatenate(
        [src, jnp.zeros((E_PAD - E,), jnp.int32)]).reshape(ER, K)
    dst2 = jnp.concatenate(
        [dst, jnp.full((E_PAD - E,), N, jnp.int32)]).reshape(ER, K)

    degp = _get_deg_counts()(dst2,
                             jnp.ones((K, 128), jnp.float32),
                             jnp.zeros((N_PAD, 128), jnp.float32))
    x_pad = jnp.pad(x, ((0, N_PAD - N), (0, 0)))

    u3, dinv = pl.pallas_call(
        _prep_body,
        grid=(GRID,),
        in_specs=[
            pl.BlockSpec((NC, BLK, 128), lambda i: (0, i, 0)),
            pl.BlockSpec((BLK, F_IN), lambda i: (i, 0)),
        ],
        out_specs=[
            pl.BlockSpec((2, BLK, 128), lambda i: (0, i, 0)),
            pl.BlockSpec((BLK, 1), lambda i: (i, 0)),
        ],
        out_shape=[
            jax.ShapeDtypeStruct((2, N_PAD, 128), jnp.float32),
            jax.ShapeDtypeStruct((N_PAD, 1), jnp.float32),
        ],
    )(degp, x_pad)

    s2 = _make_scatter(2)(u3.reshape(2 * N_PAD, 128), src2, dst2)

    wpa, wsa, wpc, wsc, bpa, bsa, bpc, bsc = pl.pallas_call(
        _wts_body,
        out_shape=[
            jax.ShapeDtypeStruct((HID, 128), jnp.float32),
            jax.ShapeDtypeStruct((HID, 128), jnp.float32),
            jax.ShapeDtypeStruct((HID, 16), jnp.float32),
            jax.ShapeDtypeStruct((HID, 16), jnp.float32),
            jax.ShapeDtypeStruct((1, 128), jnp.float32),
            jax.ShapeDtypeStruct((1, 128), jnp.float32),
            jax.ShapeDtypeStruct((1, 16), jnp.float32),
            jax.ShapeDtypeStruct((1, 16), jnp.float32),
        ],
    )(Wp1, Ws1, Wa, Wc, bp1.reshape(1, DV), bs1.reshape(1, DV))

    w3 = pl.pallas_call(
        _mid_body,
        grid=(GRID,),
        in_specs=[
            pl.BlockSpec((2, BLK, 128), lambda i: (0, i, 0)),
            pl.BlockSpec((BLK, 1), lambda i: (i, 0)),
            _full((F_IN, HID)),
            _full((1, HID)),
            _full((F_IN, HID)),
            _full((1, HID)),
            _full((HID, 128)),
            _full((HID, 128)),
            _full((HID, 16)),
            _full((HID, 16)),
        ],
        out_specs=pl.BlockSpec((3, BLK, 128), lambda i: (0, i, 0)),
        out_shape=jax.ShapeDtypeStruct((3, N_PAD, 128), jnp.float32),
    )(s2, dinv, Wp0, bp0.reshape(1, HID), Ws0, bs0.reshape(1, HID),
      wpa, wsa, wpc, wsc)

    s4 = _get_scatter_l2()(w3.reshape(3 * N_PAD, 128), src2, dst2,
                           jnp.zeros((N_PAD, 128), jnp.float32))

    out = pl.pallas_call(
        _fin_body,
        grid=(GRID,),
        in_specs=[
            pl.BlockSpec((4, BLK, 128), lambda i: (0, i, 0)),
            pl.BlockSpec((BLK, 1), lambda i: (i, 0)),
            _full((1, 128)),
            _full((1, 128)),
            _full((1, 16)),
            _full((1, 16)),
            _full((128, 1)),
            _full((1, 16)),
        ],
        out_specs=pl.BlockSpec((BLK, 16), lambda i: (i, 0)),
        out_shape=jax.ShapeDtypeStruct((N_PAD, 16), jnp.float32),
    )(s4, dinv, bpa, bsa, bpc, bsc, va.reshape(128, 1), bc.reshape(1, 16))

    return out[:N]


# speedup vs baseline: 1.1226x; 1.0254x over previous
import functools

import jax
import jax.numpy as jnp
from jax import lax
from jax.experimental import pallas as pl
from jax.experimental.pallas import tpu as pltpu
from jax.experimental.pallas import tpu_sc as plsc

N = 10000
N_PAD = 10240
E = 160000
E_PAD = 163840
ER = E_PAD // 128
F_IN = 256
HID = 512
DV = 256

NC = 2
NS = 16
LANES = 16
K = 128

ROWS_PER_TILE = N_PAD // NS

DROWS = ER // (NC * NS)
KA = 64
CH = E_PAD // (NS * 2 * KA)
NB = 4
LOOK = 3


def _mesh():
    return plsc.VectorSubcoreMesh(core_axis_name="c", subcore_axis_name="s")


@functools.cache
def _get_deg_counts():
    return functools.partial(
        pl.kernel,
        out_type=jax.ShapeDtypeStruct((NC, N_PAD, 128), jnp.float32),
        mesh=_mesh(),
        scratch_types=[
            pltpu.VMEM((DROWS, K), jnp.int32),
            pltpu.VMEM((K, 128), jnp.float32),
            pltpu.VMEM_SHARED((N_PAD, 128), jnp.float32),
            pltpu.SemaphoreType.DMA,
        ],
    )(_deg_body)


def _deg_body(dst_hbm, ones_hbm, zeros_hbm, out_hbm, dst_v, ones_v, acc, sem):
    c = lax.axis_index("c")
    s = lax.axis_index("s")
    r0 = s * ROWS_PER_TILE
    pltpu.sync_copy(zeros_hbm.at[pl.ds(r0, ROWS_PER_TILE)],
                    acc.at[pl.ds(r0, ROWS_PER_TILE)])
    pltpu.sync_copy(ones_hbm, ones_v)
    pltpu.sync_copy(dst_hbm.at[pl.ds((c * NS + s) * DROWS, DROWS)], dst_v)
    plsc.subcore_barrier()

    def fire(i, carry):
        pltpu.async_copy(ones_v, acc.at[dst_v.at[i]], sem, add=True)
        return carry

    lax.fori_loop(0, DROWS, fire, 0)

    def drain(i, carry):
        pltpu.make_async_copy(ones_v, acc.at[dst_v.at[i]], sem).wait()
        return carry

    lax.fori_loop(0, DROWS, drain, 0)
    plsc.subcore_barrier()
    pltpu.sync_copy(acc.at[pl.ds(r0, ROWS_PER_TILE)],
                    out_hbm.at[c, pl.ds(r0, ROWS_PER_TILE)])


def _edge_block(u_hbm, src_hbm, dst_hbm, acc,
                src_v, dst_v, ibuf, dbuf, rows, gsem, ssem, e0, qn):
    pltpu.sync_copy(src_hbm.at[pl.ds(e0, CH // 2)], src_v)
    pltpu.sync_copy(dst_hbm.at[pl.ds(e0, CH // 2)], dst_v)

    def prep(buf, ev, row, col, add):
        for t in range(KA // LANES):
            buf[pl.ds(t * LANES, LANES)] = (
                ev[row, pl.ds(col + t * LANES, LANES)] + add)

    for j in range(LOOK):
        prep(ibuf[j], src_v, j // 2, KA * (j % 2), qn)
        pltpu.async_copy(u_hbm.at[ibuf[j]], rows[j], gsem[j])

    def quad(g, carry):
        for j in range(NB):
            ci = NB * g + j
            row = 2 * g + j // 2
            col = KA * (j % 2)
            pltpu.make_async_copy(u_hbm.at[ibuf[j]], rows[j], gsem[j]).wait()
            prep(dbuf[j], dst_v, row, col, 0)
            pltpu.async_copy(rows[j], acc.at[dbuf[j]], ssem[j], add=True)
            jn = (j + LOOK) % NB
            rown = 2 * g + (j + LOOK) // 2
            coln = KA * ((j + LOOK) % 2)

            @pl.when(ci + LOOK < CH)
            def _(ci=ci, jn=jn, rown=rown, coln=coln):
                @pl.when(ci + LOOK >= NB)
                def _():
                    pltpu.make_async_copy(rows[jn], acc.at[dbuf[jn]],
                                          ssem[jn]).wait()
                prep(ibuf[jn], src_v, rown, coln, qn)
                pltpu.async_copy(u_hbm.at[ibuf[jn]], rows[jn], gsem[jn])
        return carry

    lax.fori_loop(0, CH // NB, quad, 0)
    for j in range(NB):
        pltpu.make_async_copy(rows[j], acc.at[dbuf[j]], ssem[j]).wait()


def _sc_scratch():
    return (
        [
            pltpu.VMEM((CH // 2, 128), jnp.int32),
            pltpu.VMEM((CH // 2, 128), jnp.int32),
        ]
        + [pltpu.VMEM((KA,), jnp.int32) for _ in range(NB)]
        + [pltpu.VMEM((KA,), jnp.int32) for _ in range(NB)]
        + [pltpu.VMEM((KA, 128), jnp.float32) for _ in range(NB)]
        + [pltpu.VMEM_SHARED((N_PAD, 128), jnp.float32)]
        + [pltpu.SemaphoreType.DMA for _ in range(2 * NB)]
    )


@functools.cache
def _make_scatter(Q):
    P = Q // NC

    @functools.partial(
        pl.kernel,
        out_type=jax.ShapeDtypeStruct((Q, N_PAD, 128), jnp.float32),
        mesh=_mesh(),
        scratch_types=_sc_scratch(),
    )
    def _scatter(u_hbm, src_hbm, dst_hbm, out_hbm,
                 src_v, dst_v, ib0, ib1, ib2, ib3, db0, db1, db2, db3,
                 r0_v, r1_v, r2_v, r3_v, acc,
                 g0, g1, g2, g3, s0, s1, s2, s3):
        ibuf = [ib0, ib1, ib2, ib3]
        dbuf = [db0, db1, db2, db3]
        rows = [r0_v, r1_v, r2_v, r3_v]
        gsem = [g0, g1, g2, g3]
        ssem = [s0, s1, s2, s3]
        c = lax.axis_index("c")
        s = lax.axis_index("s")
        r0 = s * ROWS_PER_TILE
        for p in range(P):
            q = p * NC + c
            qn = q * N_PAD
            pltpu.sync_copy(u_hbm.at[pl.ds(qn + r0, ROWS_PER_TILE)],
                            acc.at[pl.ds(r0, ROWS_PER_TILE)])
            plsc.subcore_barrier()
            for h in range(2):
                _edge_block(u_hbm, src_hbm, dst_hbm, acc, src_v, dst_v,
                            ibuf, dbuf, rows, gsem, ssem,
                            s * CH + h * (CH // 2), qn)
            plsc.subcore_barrier()
            pltpu.sync_copy(acc.at[pl.ds(r0, ROWS_PER_TILE)],
                            out_hbm.at[q, pl.ds(r0, ROWS_PER_TILE)])

    return _scatter


@functools.cache
def _get_scatter_l2():
    @functools.partial(
        pl.kernel,
        out_type=jax.ShapeDtypeStruct((4, N_PAD, 128), jnp.float32),
        mesh=_mesh(),
        scratch_types=_sc_scratch(),
    )
    def _scatter3(w_hbm, src_hbm, dst_hbm, zeros_hbm, out_hbm,
                  src_v, dst_v, ib0, ib1, ib2, ib3, db0, db1, db2, db3,
                  r0_v, r1_v, r2_v, r3_v, acc,
                  g0, g1, g2, g3, s0, s1, s2, s3):
        ibuf = [ib0, ib1, ib2, ib3]
        dbuf = [db0, db1, db2, db3]
        rows = [r0_v, r1_v, r2_v, r3_v]
        gsem = [g0, g1, g2, g3]
        ssem = [s0, s1, s2, s3]
        c = lax.axis_index("c")
        s = lax.axis_index("s")
        r0 = s * ROWS_PER_TILE
        qn = c * N_PAD
        pltpu.sync_copy(w_hbm.at[pl.ds(qn + r0, ROWS_PER_TILE)],
                        acc.at[pl.ds(r0, ROWS_PER_TILE)])
        plsc.subcore_barrier()
        for h in range(2):
            _edge_block(w_hbm, src_hbm, dst_hbm, acc, src_v, dst_v,
                        ibuf, dbuf, rows, gsem, ssem,
                        s * CH + h * (CH // 2), qn)
        plsc.subcore_barrier()
        pltpu.sync_copy(acc.at[pl.ds(r0, ROWS_PER_TILE)],
                        out_hbm.at[c, pl.ds(r0, ROWS_PER_TILE)])
        qn2 = (2 + c) * N_PAD

        @pl.when(c == 0)
        def _():
            pltpu.sync_copy(w_hbm.at[pl.ds(2 * N_PAD + r0, ROWS_PER_TILE)],
                            acc.at[pl.ds(r0, ROWS_PER_TILE)])

        @pl.when(c == 1)
        def _():
            pltpu.sync_copy(zeros_hbm.at[pl.ds(r0, ROWS_PER_TILE)],
                            acc.at[pl.ds(r0, ROWS_PER_TILE)])

        plsc.subcore_barrier()
        _edge_block(w_hbm, src_hbm, dst_hbm, acc, src_v, dst_v,
                    ibuf, dbuf, rows, gsem, ssem,
                    (c * NS + s) * (CH // 2), qn2)
        plsc.subcore_barrier()
        pltpu.sync_copy(acc.at[pl.ds(r0, ROWS_PER_TILE)],
                        out_hbm.at[2 + c, pl.ds(r0, ROWS_PER_TILE)])

    return _scatter3


BLK = 1024
GRID = N_PAD // BLK


def _prep_body(degp_ref, x_ref, u_ref, dinv_ref):
    deg = 1.0 + degp_ref[0, :, 0:1] + degp_ref[1, :, 0:1]
    dinv = lax.rsqrt(deg)
    dinv_ref[...] = dinv
    u = dinv * x_ref[...]
    u_ref[0] = u[:, :128]
    u_ref[1] = u[:, 128:]


def _wts_body(wp1_ref, ws1_ref, wa_ref, wc_ref, bp1_ref, bs1_ref,
              wpa_ref, wsa_ref, wpc_ref, wsc_ref,
              bpa_ref, bsa_ref, bpc_ref, bsc_ref):
    wpa_ref[...] = jnp.dot(wp1_ref[...], wa_ref[...])
    wsa_ref[...] = jnp.dot(ws1_ref[...], wa_ref[...])
    wpc_ref[...] = jnp.dot(wp1_ref[...], wc_ref[...])
    wsc_ref[...] = jnp.dot(ws1_ref[...], wc_ref[...])
    bpa_ref[...] = jnp.dot(bp1_ref[...], wa_ref[...])
    bsa_ref[...] = jnp.dot(bs1_ref[...], wa_ref[...])
    bpc_ref[...] = jnp.dot(bp1_ref[...], wc_ref[...])
    bsc_ref[...] = jnp.dot(bs1_ref[...], wc_ref[...])


def _mid_body(s_ref, dinv_ref, wp0_ref, bp0_ref, ws0_ref, bs0_ref,
              wpa_ref, wsa_ref, wpc_ref, wsc_ref, w_ref):
    dinv = dinv_ref[...]
    a0 = dinv * jnp.concatenate([s_ref[0], s_ref[1]], axis=1)
    hp = jnp.maximum(jnp.dot(a0, wp0_ref[...]) + bp0_ref[...], 0.0)
    hs = jnp.maximum(jnp.dot(a0, ws0_ref[...]) + bs0_ref[...], 0.0)
    w_ref[0] = dinv * jnp.dot(hp, wpa_ref[...])
    w_ref[1] = dinv * jnp.dot(hs, wsa_ref[...])
    wc2 = jnp.concatenate(
        [dinv * jnp.dot(hp, wpc_ref[...]),
         dinv * jnp.dot(hs, wsc_ref[...]),
         jnp.zeros((hp.shape[0], 96), jnp.float32)], axis=1)
    w_ref[2] = wc2
    w_ref[3] = wc2


def _fin_body(s_ref, dinv_ref, bpa_ref, bsa_ref, bpc_ref, bsc_ref,
              va_ref, bc_ref, out_ref):
    dinv = dinv_ref[...]
    muap = dinv * s_ref[0] + bpa_ref[...]
    muas = dinv * s_ref[1] + bsa_ref[...]
    sp = jnp.dot(jnp.tanh(muap), va_ref[...])
    ss = jnp.dot(jnp.tanh(muas), va_ref[...])
    mc = dinv * (s_ref[2] + s_ref[3])
    mcp = mc[:, 0:16] + bpc_ref[...]
    mcs = mc[:, 16:32] + bsc_ref[...]
    m = jnp.maximum(sp, ss)
    e0 = jnp.exp(sp - m)
    e1 = jnp.exp(ss - m)
    out_ref[...] = (e0 * mcp + e1 * mcs) / (e0 + e1) + bc_ref[...]


def _full(shape):
    nd = len(shape)
    return pl.BlockSpec(shape, lambda i, _nd=nd: (0,) * _nd)


def kernel(x, edge_index, Wp0, bp0, Wp1, bp1, Wp2, bp2,
           Ws0, bs0, Ws1, bs1, Ws2, bs2, Wa, va, Wc, bc):
    src = edge_index[0]
    dst = edge_index[1]
    src2 = jnp.concatenate(
        [src, jnp.zeros((E_PAD - E,), jnp.int32)]).reshape(ER, K)
    dst2 = jnp.concatenate(
        [dst, jnp.full((E_PAD - E,), N, jnp.int32)]).reshape(ER, K)

    degp = _get_deg_counts()(dst2,
                             jnp.ones((K, 128), jnp.float32),
                             jnp.zeros((N_PAD, 128), jnp.float32))
    x_pad = jnp.pad(x, ((0, N_PAD - N), (0, 0)))

    u3, dinv = pl.pallas_call(
        _prep_body,
        grid=(GRID,),
        in_specs=[
            pl.BlockSpec((NC, BLK, 128), lambda i: (0, i, 0)),
            pl.BlockSpec((BLK, F_IN), lambda i: (i, 0)),
        ],
        out_specs=[
            pl.BlockSpec((2, BLK, 128), lambda i: (0, i, 0)),
            pl.BlockSpec((BLK, 1), lambda i: (i, 0)),
        ],
        out_shape=[
            jax.ShapeDtypeStruct((2, N_PAD, 128), jnp.float32),
            jax.ShapeDtypeStruct((N_PAD, 1), jnp.float32),
        ],
    )(degp, x_pad)

    s2 = _make_scatter(2)(u3.reshape(2 * N_PAD, 128), src2, dst2)

    wpa, wsa, wpc, wsc, bpa, bsa, bpc, bsc = pl.pallas_call(
        _wts_body,
        out_shape=[
            jax.ShapeDtypeStruct((HID, 128), jnp.float32),
            jax.ShapeDtypeStruct((HID, 128), jnp.float32),
            jax.ShapeDtypeStruct((HID, 16), jnp.float32),
            jax.ShapeDtypeStruct((HID, 16), jnp.float32),
            jax.ShapeDtypeStruct((1, 128), jnp.float32),
            jax.ShapeDtypeStruct((1, 128), jnp.float32),
            jax.ShapeDtypeStruct((1, 16), jnp.float32),
            jax.ShapeDtypeStruct((1, 16), jnp.float32),
        ],
    )(Wp1, Ws1, Wa, Wc, bp1.reshape(1, DV), bs1.reshape(1, DV))

    w3 = pl.pallas_call(
        _mid_body,
        grid=(GRID,),
        in_specs=[
            pl.BlockSpec((2, BLK, 128), lambda i: (0, i, 0)),
            pl.BlockSpec((BLK, 1), lambda i: (i, 0)),
            _full((F_IN, HID)),
            _full((1, HID)),
            _full((F_IN, HID)),
            _full((1, HID)),
            _full((HID, 128)),
            _full((HID, 128)),
            _full((HID, 16)),
            _full((HID, 16)),
        ],
        out_specs=pl.BlockSpec((4, BLK, 128), lambda i: (0, i, 0)),
        out_shape=jax.ShapeDtypeStruct((4, N_PAD, 128), jnp.float32),
    )(s2, dinv, Wp0, bp0.reshape(1, HID), Ws0, bs0.reshape(1, HID),
      wpa, wsa, wpc, wsc)

    s4 = _get_scatter_l2()(w3.reshape(4 * N_PAD, 128), src2, dst2,
                           jnp.zeros((N_PAD, 128), jnp.float32))

    out = pl.pallas_call(
        _fin_body,
        grid=(GRID,),
        in_specs=[
            pl.BlockSpec((4, BLK, 128), lambda i: (0, i, 0)),
            pl.BlockSpec((BLK, 1), lambda i: (i, 0)),
            _full((1, 128)),
            _full((1, 128)),
            _full((1, 16)),
            _full((1, 16)),
            _full((128, 1)),
            _full((1, 16)),
        ],
        out_specs=pl.BlockSpec((BLK, 16), lambda i: (i, 0)),
        out_shape=jax.ShapeDtypeStruct((N_PAD, 16), jnp.float32),
    )(s4, dinv, bpa, bsa, bpc, bsc, va.reshape(128, 1), bc.reshape(1, 16))

    return out[:N]
